# baseline (speedup 1.0000x reference)
import jax
import jax.numpy as jnp
from jax import lax
from jax.experimental import pallas as pl
from jax.experimental.pallas import tpu as pltpu
from jax.experimental.pallas import tpu_sc as plsc

N = 10000
E = 320000
F = 128
NC = 2
NS = 16
NW = NC * NS
N_PAD = 10240
ROWS_PER_SUB = N_PAD // NS
CHUNK = 128
NCH = 80
E_PAD = NW * NCH * CHUNK
G = E_PAD // CHUNK
BLKCH = 8
NBLK = NCH // BLKCH
L = 16


def _agg_body(with_deg):
    def body(*refs):
        if with_deg:
            (feat_hbm, src_hbm, dst_hbm, zeros_hbm,
             out0_hbm, out1_hbm, deg_hbm,
             srcbuf, dstbuf, rows_a, rows_b, hist,
             acc, sem_ga, sem_gb, sem_sa, sem_sb) = refs
        else:
            (feat_hbm, src_hbm, dst_hbm, zeros_hbm,
             out0_hbm, out1_hbm,
             srcbuf, dstbuf, rows_a, rows_b,
             acc, sem_ga, sem_gb, sem_sa, sem_sb) = refs
        c = lax.axis_index("c")
        s = lax.axis_index("s")
        wid = s * NC + c
        r0 = s * ROWS_PER_SUB
        cb = wid * NCH

        pltpu.sync_copy(zeros_hbm.at[pl.ds(r0, ROWS_PER_SUB)],
                        acc.at[pl.ds(r0, ROWS_PER_SUB)])
        if with_deg:
            def zstep(i, carry):
                hist[pl.ds(i * L, L)] = jnp.zeros((L,), jnp.float32)
                return carry
            lax.fori_loop(0, N_PAD // L, zstep, 0)
        plsc.subcore_barrier()

        ones = jnp.ones((L,), jnp.float32)

        def do_hist(j):
            for k in range(CHUNK // L):
                plsc.addupdate_scatter(
                    hist, [dstbuf[j, pl.ds(k * L, L)]], ones)

        def blk_body(b, carry):
            bc = cb + b * BLKCH
            pltpu.sync_copy(src_hbm.at[pl.ds(bc, BLKCH)], srcbuf)
            pltpu.sync_copy(dst_hbm.at[pl.ds(bc, BLKCH)], dstbuf)
            rows = (rows_a, rows_b)
            gsem = (sem_ga, sem_gb)
            ssem = (sem_sa, sem_sb)
            gat = [None] * BLKCH
            sca = [None] * BLKCH
            gat[0] = pltpu.async_copy(feat_hbm.at[srcbuf.at[0]],
                                      rows[0], gsem[0])
            for j in range(BLKCH):
                p = j % 2
                q = (j + 1) % 2
                if j + 1 < BLKCH:
                    if j >= 1:
                        sca[j - 1].wait()
                    gat[j + 1] = pltpu.async_copy(
                        feat_hbm.at[srcbuf.at[j + 1]], rows[q], gsem[q])
                gat[j].wait()
                sca[j] = pltpu.async_copy(rows[p], acc.at[dstbuf.at[j]],
                                          ssem[p], add=True)
                if with_deg:
                    do_hist(j)
            sca[BLKCH - 2].wait()
            sca[BLKCH - 1].wait()
            return carry

        lax.fori_loop(0, NBLK, blk_body, 0)
        plsc.subcore_barrier()

        @pl.when(c == 0)
        def _():
            pltpu.sync_copy(acc.at[pl.ds(r0, ROWS_PER_SUB)],
                            out0_hbm.at[pl.ds(r0, ROWS_PER_SUB)])

        @pl.when(c == 1)
        def _():
            pltpu.sync_copy(acc.at[pl.ds(r0, ROWS_PER_SUB)],
                            out1_hbm.at[pl.ds(r0, ROWS_PER_SUB)])

        if with_deg:
            pltpu.sync_copy(hist, deg_hbm.at[wid])
    return body


def _make_agg(with_deg):
    scratch = [
        pltpu.VMEM((BLKCH, CHUNK), jnp.int32),
        pltpu.VMEM((BLKCH, CHUNK), jnp.int32),
        pltpu.VMEM((CHUNK, F), jnp.float32),
        pltpu.VMEM((CHUNK, F), jnp.float32),
    ]
    out_type = [jax.ShapeDtypeStruct((N_PAD, F), jnp.float32),
                jax.ShapeDtypeStruct((N_PAD, F), jnp.float32)]
    if with_deg:
        scratch += [pltpu.VMEM((N_PAD,), jnp.float32)]
        out_type += [jax.ShapeDtypeStruct((NW, N_PAD), jnp.float32)]
    scratch += [pltpu.VMEM_SHARED((N_PAD, F), jnp.float32),
                pltpu.SemaphoreType.DMA,
                pltpu.SemaphoreType.DMA,
                pltpu.SemaphoreType.DMA,
                pltpu.SemaphoreType.DMA]
    return pl.kernel(
        _agg_body(with_deg),
        out_type=tuple(out_type),
        mesh=plsc.VectorSubcoreMesh(core_axis_name="c", subcore_axis_name="s"),
        scratch_types=scratch,
        compiler_params=pltpu.CompilerParams(needs_layout_passes=False),
        name="sage_agg_deg" if with_deg else "sage_agg",
    )


_agg_deg_call = _make_agg(True)
_agg_call = _make_agg(False)

BLK = 1280


def _combine_body(relu):
    def body(x_ref, p0_ref, p1_ref, dp_ref, ws_ref, wn_ref, b_ref, out_ref):
        agg = p0_ref[...] + p1_ref[...]
        deg = jnp.sum(dp_ref[...], axis=0).reshape(BLK, 1)
        mean = agg * (1.0 / jnp.maximum(deg, 1.0))
        y = (jnp.dot(x_ref[...], ws_ref[...],
                     preferred_element_type=jnp.float32)
             + jnp.dot(mean, wn_ref[...], preferred_element_type=jnp.float32)
             + b_ref[...])
        out_ref[...] = jnp.maximum(y, 0.0) if relu else y
    return body


def _combine(x, p0, p1, degparts, Ws, Wn, b, relu):
    return pl.pallas_call(
        _combine_body(relu),
        out_shape=jax.ShapeDtypeStruct((N_PAD, F), jnp.float32),
        grid=(N_PAD // BLK,),
        in_specs=[
            pl.BlockSpec((BLK, F), lambda i: (i, 0)),
            pl.BlockSpec((BLK, F), lambda i: (i, 0)),
            pl.BlockSpec((BLK, F), lambda i: (i, 0)),
            pl.BlockSpec((NW, BLK), lambda i: (0, i)),
            pl.BlockSpec((F, F), lambda i: (0, 0)),
            pl.BlockSpec((F, F), lambda i: (0, 0)),
            pl.BlockSpec((1, F), lambda i: (0, 0)),
        ],
        out_specs=pl.BlockSpec((BLK, F), lambda i: (i, 0)),
    )(x, p0, p1, degparts, Ws, Wn, b.reshape(1, F))


def kernel(x, edge_index, W_self1, W_neigh1, b1, W_self2, W_neigh2, b2):
    src = edge_index[0]
    dst = edge_index[1]
    npad = E_PAD - E
    src2d = jnp.concatenate(
        [src, jnp.zeros((npad,), jnp.int32)]).reshape(G, CHUNK)
    dst2d = jnp.concatenate(
        [dst, jnp.full((npad,), N, jnp.int32)]).reshape(G, CHUNK)
    x_pad = jnp.pad(x, ((0, N_PAD - N), (0, 0)))
    zeros_hbm = jnp.zeros((N_PAD, F), jnp.float32)

    p0, p1, degparts = _agg_deg_call(x_pad, src2d, dst2d, zeros_hbm)
    h = _combine(x_pad, p0, p1, degparts, W_self1, W_neigh1, b1, relu=True)
    q0, q1 = _agg_call(h, src2d, dst2d, zeros_hbm)
    out = _combine(h, q0, q1, degparts, W_self2, W_neigh2, b2, relu=False)
    return out[:N]
